# Initial kernel scaffold; baseline (speedup 1.0000x reference)
import jax
import jax.numpy as jnp
from jax.experimental import pallas as pl


def kernel(pred, target):
    raise NotImplementedError("write your pallas kernel here")


# speedup vs baseline: 1.8057x; 1.8057x over previous
import functools
import jax
import jax.numpy as jnp
from jax import lax
from jax.experimental import pallas as pl

_EPS = 0.1
_V = 100000
_N = 2048
_FILL = _EPS / (_V - 2)

_R = 256
_C = 2048
_NC = (_V + _C - 1) // _C


def _stats_kernel(pred_ref, tgt_ref, m_ref, s_ref, tot_ref, tv_ref):
    j = pl.program_id(1)

    @pl.when(j == 0)
    def _init():
        m_ref[...] = jnp.full((_R, 1), -1e30, jnp.float32)
        s_ref[...] = jnp.zeros((_R, 1), jnp.float32)
        tot_ref[...] = jnp.zeros((_R, 1), jnp.float32)
        tv_ref[...] = jnp.zeros((_R, 1), jnp.float32)

    x = pred_ref[...]
    cols = lax.broadcasted_iota(jnp.int32, (_R, _C), 1) + j * _C
    valid = cols < _V
    xm = jnp.where(valid, x, -jnp.inf)
    xs = jnp.where(valid, x, 0.0)

    t = tgt_ref[...]
    tv_ref[...] += jnp.sum(jnp.where(cols == t, xs, 0.0), axis=1, keepdims=True)
    tot_ref[...] += jnp.sum(xs, axis=1, keepdims=True)

    cmax = jnp.max(xm, axis=1, keepdims=True)
    m_old = m_ref[...]
    m_new = jnp.maximum(m_old, cmax)
    s_ref[...] = s_ref[...] * jnp.exp(m_old - m_new) + jnp.sum(
        jnp.exp(xm - m_new), axis=1, keepdims=True)
    m_ref[...] = m_new


def _combine_kernel(m_ref, s_ref, tot_ref, tv_ref, tgt_ref, out_ref):
    logz = m_ref[...] + jnp.log(s_ref[...])
    s_row = tot_ref[...] - _V * logz
    logp_t = tv_ref[...] - logz
    loss = -(_FILL * s_row + (1.0 - _EPS - _FILL) * logp_t)
    mask = tgt_ref[...] != 0
    loss_sum = jnp.sum(jnp.where(mask, loss, 0.0), keepdims=True).reshape(1, 1)
    cnt = jnp.sum(mask.astype(jnp.float32), keepdims=True).reshape(1, 1)
    out_ref[...] = jnp.where(cnt > 0, loss_sum / jnp.maximum(cnt, 1.0), 0.0)


def kernel(pred, target):
    tgt2 = target.reshape(_N, 1)
    m, s, tot, tv = pl.pallas_call(
        _stats_kernel,
        grid=(_N // _R, _NC),
        in_specs=[
            pl.BlockSpec((_R, _C), lambda i, j: (i, j)),
            pl.BlockSpec((_R, 1), lambda i, j: (i, 0)),
        ],
        out_specs=[pl.BlockSpec((_R, 1), lambda i, j: (i, 0))] * 4,
        out_shape=[jax.ShapeDtypeStruct((_N, 1), jnp.float32)] * 4,
    )(pred, tgt2)

    out = pl.pallas_call(
        _combine_kernel,
        out_shape=jax.ShapeDtypeStruct((1, 1), jnp.float32),
    )(m.reshape(16, 128), s.reshape(16, 128), tot.reshape(16, 128),
      tv.reshape(16, 128), target.reshape(16, 128))
    return out[0, 0]
